# Initial kernel scaffold; baseline (speedup 1.0000x reference)
import jax
import jax.numpy as jnp
from jax.experimental import pallas as pl


def kernel(x):
    raise NotImplementedError("write your pallas kernel here")


# speedup vs baseline: 6.5966x; 6.5966x over previous
import jax
import jax.numpy as jnp
from jax.experimental import pallas as pl
from jax.experimental.pallas import tpu as pltpu

_C, _H, _W = 512, 256, 256
_BC = 8


def _corner_pool_kernel(x_ref, o_ref):
    x = x_ref[...]
    neg = jnp.float32(float("-inf"))

    step = 1
    while step < _H:
        pad = jnp.full((_BC, step, _W), neg, dtype=x.dtype)
        shifted = jnp.concatenate([pad, x[:, : _H - step, :]], axis=1)
        x = jnp.maximum(x, shifted)
        step *= 2

    step = 1
    while step < _W:
        pad = jnp.full((_BC, _H, step), neg, dtype=x.dtype)
        shifted = jnp.concatenate([pad, x[:, :, : _W - step]], axis=2)
        x = jnp.maximum(x, shifted)
        step *= 2

    o_ref[...] = x + x


@jax.jit
def kernel(x):
    return pl.pallas_call(
        _corner_pool_kernel,
        grid=(_C // _BC,),
        in_specs=[pl.BlockSpec((_BC, _H, _W), lambda i: (i, 0, 0))],
        out_specs=pl.BlockSpec((_BC, _H, _W), lambda i: (i, 0, 0)),
        out_shape=jax.ShapeDtypeStruct((_C, _H, _W), x.dtype),
        compiler_params=pltpu.CompilerParams(
            dimension_semantics=("parallel",),
        ),
    )(x)
